# baseline (speedup 1.0000x reference)
import jax
import jax.numpy as jnp
from jax import lax
from jax.experimental import pallas as pl
from jax.experimental.pallas import tpu as pltpu
from jax.experimental.pallas import tpu_sc as plsc

_STEPS = 1024
_CHANNELS = 256
_N = 65536
_NC = 2
_NS = 16
_NW = _NC * _NS
_PER_W = _N // _NW
_CHUNK = 64
_NCHUNK = _PER_W // _CHUNK
_NPAIR = _NCHUNK // 2
_L = 16
_WORDS = _CHANNELS // 2


def _body(t_hbm, cbits_hbm, out_hbm,
          t_v, idx0_v, idx1_v, w1_v,
          r0a, r1a, r0b, r1b, oa, ob, spt,
          sg0a, sg1a, sg0b, sg1b, soa, sob):
    sid = lax.axis_index("s")
    wid = sid * _NC + lax.axis_index("c")
    base = wid * _PER_W

    rows_per_tile = _STEPS // _NS
    pltpu.sync_copy(cbits_hbm.at[pl.ds(sid * rows_per_tile, rows_per_tile)],
                    spt.at[pl.ds(sid * rows_per_tile, rows_per_tile)])
    pltpu.sync_copy(t_hbm.at[pl.ds(base, _PER_W)], t_v)

    def idx_body(g, carry):
        t16 = t_v[pl.ds(g * _L, _L)]
        pos = t16 * float(_STEPS - 1)
        i0 = lax.convert_element_type(pos, jnp.int32)
        i0 = jnp.maximum(jnp.minimum(i0, _STEPS - 2), 0)
        frac = pos - lax.convert_element_type(i0, jnp.float32)
        idx0_v[pl.ds(g * _L, _L)] = i0
        idx1_v[pl.ds(g * _L, _L)] = i0 + 1
        w1_v[pl.ds(g * _L, _L)] = frac
        return carry

    lax.fori_loop(0, _PER_W // _L, idx_body, 0)
    plsc.subcore_barrier()

    def g_descs(k, r0, r1, s0, s1):
        d0 = pltpu.make_async_copy(
            spt.at[idx0_v.at[pl.ds(k * _CHUNK, _CHUNK)]], r0, s0)
        d1 = pltpu.make_async_copy(
            spt.at[idx1_v.at[pl.ds(k * _CHUNK, _CHUNK)]], r1, s1)
        return d0, d1

    def g_issue(k, r0, r1, s0, s1):
        d0, d1 = g_descs(k, r0, r1, s0, s1)
        d0.start()
        d1.start()

    def g_wait(k, r0, r1, s0, s1):
        d0, d1 = g_descs(k, r0, r1, s0, s1)
        d0.wait()
        d1.wait()

    def o_desc(k, o, so):
        return pltpu.make_async_copy(
            o, out_hbm.at[pl.ds(base + k * _CHUNK, _CHUNK)], so)

    def combine(k, r0, r1, o):
        @plsc.parallel_loop(0, _CHUNK, unroll=4)
        def t_body(j):
            w16 = w1_v[pl.ds(k * _CHUNK + (j // _L) * _L, _L)]
            dn = lax.GatherDimensionNumbers(
                offset_dims=(), collapsed_slice_dims=(0,),
                start_index_map=(0,))
            w1 = lax.gather(
                w16, jnp.full((_L, 1), j % _L, jnp.int32), dn, (1,),
                mode=lax.GatherScatterMode.PROMISE_IN_BOUNDS)
            w1b = plsc.pack(w1, w1, format=plsc.PackFormat.INTERLEAVED)
            for u in range(_CHANNELS // 32):
                if u % 4 != 3:
                    a = plsc.bitcast(r0[j, pl.ds(u * _L, _L)], jnp.bfloat16)
                    b = plsc.bitcast(r1[j, pl.ds(u * _L, _L)], jnp.bfloat16)
                    ob16 = a + w1b * (b - a)
                    oe, oo = plsc.unpack(
                        ob16, format=plsc.PackFormat.INTERLEAVED)
                else:
                    a = plsc.bitcast(r0[j, pl.ds(u * _L, _L)], jnp.bfloat16)
                    b = plsc.bitcast(r1[j, pl.ds(u * _L, _L)], jnp.bfloat16)
                    ae, ao = plsc.unpack(a, format=plsc.PackFormat.INTERLEAVED)
                    be, bo = plsc.unpack(b, format=plsc.PackFormat.INTERLEAVED)
                    oe = ae + w1 * (be - ae)
                    oo = ao + w1 * (bo - ao)
                o[j, pl.ds(u * 32, _L)] = oe
                o[j, pl.ds(u * 32 + _L, _L)] = oo

    g_issue(0, r0a, r1a, sg0a, sg1a)

    def pair_body(p, carry):
        k0 = 2 * p
        g_wait(k0, r0a, r1a, sg0a, sg1a)
        g_issue(k0 + 1, r0b, r1b, sg0b, sg1b)

        @pl.when(p > 0)
        def _():
            o_desc(k0 - 2, oa, soa).wait()

        combine(k0, r0a, r1a, oa)
        o_desc(k0, oa, soa).start()

        g_wait(k0 + 1, r0b, r1b, sg0b, sg1b)

        @pl.when(p + 1 < _NPAIR)
        def _():
            g_issue(k0 + 2, r0a, r1a, sg0a, sg1a)

        @pl.when(p > 0)
        def _():
            o_desc(k0 - 1, ob, sob).wait()

        combine(k0 + 1, r0b, r1b, ob)
        o_desc(k0 + 1, ob, sob).start()
        return carry

    lax.fori_loop(0, _NPAIR, pair_body, 0)

    o_desc(_NCHUNK - 2, oa, soa).wait()
    o_desc(_NCHUNK - 1, ob, sob).wait()


def kernel(t, control):
    cperm = (control.astype(jnp.bfloat16)
             .reshape(_STEPS, _CHANNELS // 32, 2, _L)
             .swapaxes(-1, -2))
    cbits = lax.bitcast_convert_type(
        cperm.reshape(_STEPS, _WORDS, 2), jnp.uint32)
    mesh = plsc.VectorSubcoreMesh(core_axis_name="c", subcore_axis_name="s")
    f = pl.kernel(
        _body,
        out_type=jax.ShapeDtypeStruct((_N, _CHANNELS), jnp.float32),
        mesh=mesh,
        compiler_params=pltpu.CompilerParams(needs_layout_passes=False),
        scratch_types=[
            pltpu.VMEM((_PER_W,), jnp.float32),
            pltpu.VMEM((_PER_W,), jnp.int32),
            pltpu.VMEM((_PER_W,), jnp.int32),
            pltpu.VMEM((_PER_W,), jnp.float32),
            pltpu.VMEM((_CHUNK, _WORDS), jnp.uint32),
            pltpu.VMEM((_CHUNK, _WORDS), jnp.uint32),
            pltpu.VMEM((_CHUNK, _WORDS), jnp.uint32),
            pltpu.VMEM((_CHUNK, _WORDS), jnp.uint32),
            pltpu.VMEM((_CHUNK, _CHANNELS), jnp.float32),
            pltpu.VMEM((_CHUNK, _CHANNELS), jnp.float32),
            pltpu.VMEM_SHARED((_STEPS, _WORDS), jnp.uint32),
            pltpu.SemaphoreType.DMA,
            pltpu.SemaphoreType.DMA,
            pltpu.SemaphoreType.DMA,
            pltpu.SemaphoreType.DMA,
            pltpu.SemaphoreType.DMA,
            pltpu.SemaphoreType.DMA,
        ],
    )
    return f(t, cbits)


# speedup vs baseline: 1.0680x; 1.0680x over previous
import jax
import jax.numpy as jnp
from jax import lax
from jax.experimental import pallas as pl
from jax.experimental.pallas import tpu as pltpu
from jax.experimental.pallas import tpu_sc as plsc

_STEPS = 1024
_CHANNELS = 256
_N = 65536
_NC = 2
_NS = 16
_NW = _NC * _NS
_PER_W = _N // _NW
_CHUNK = 64
_NCHUNK = _PER_W // _CHUNK
_NPAIR = _NCHUNK // 2
_L = 16
_WORDS = _CHANNELS // 2


def _body(t_hbm, cbits_hbm, out_hbm,
          t_v, idx0_v, idx1_v, w1_v,
          r0a, r1a, r0b, r1b, oa, ob, spt,
          sg0a, sg1a, sg0b, sg1b, soa, sob):
    sid = lax.axis_index("s")
    wid = sid * _NC + lax.axis_index("c")
    base = wid * _PER_W

    rows_per_tile = _STEPS // _NS
    pltpu.sync_copy(cbits_hbm.at[pl.ds(sid * rows_per_tile, rows_per_tile)],
                    spt.at[pl.ds(sid * rows_per_tile, rows_per_tile)])
    pltpu.sync_copy(t_hbm.at[pl.ds(base, _PER_W)], t_v)

    def idx_body(g, carry):
        t16 = t_v[pl.ds(g * _L, _L)]
        pos = t16 * float(_STEPS - 1)
        i0 = lax.convert_element_type(pos, jnp.int32)
        i0 = jnp.maximum(jnp.minimum(i0, _STEPS - 2), 0)
        frac = pos - lax.convert_element_type(i0, jnp.float32)
        idx0_v[pl.ds(g * _L, _L)] = i0
        idx1_v[pl.ds(g * _L, _L)] = i0 + 1
        w1_v[pl.ds(g * _L, _L)] = frac
        return carry

    lax.fori_loop(0, _PER_W // _L, idx_body, 0)
    plsc.subcore_barrier()

    def g_descs(k, r0, r1, s0, s1):
        d0 = pltpu.make_async_copy(
            spt.at[idx0_v.at[pl.ds(k * _CHUNK, _CHUNK)]], r0, s0)
        d1 = pltpu.make_async_copy(
            spt.at[idx1_v.at[pl.ds(k * _CHUNK, _CHUNK)]], r1, s1)
        return d0, d1

    def g_issue(k, r0, r1, s0, s1):
        d0, d1 = g_descs(k, r0, r1, s0, s1)
        d0.start()
        d1.start()

    def g_wait(k, r0, r1, s0, s1):
        d0, d1 = g_descs(k, r0, r1, s0, s1)
        d0.wait()
        d1.wait()

    def o_desc(k, o, so):
        return pltpu.make_async_copy(
            o, out_hbm.at[pl.ds(base + k * _CHUNK, _CHUNK)], so)

    def combine(k, r0, r1, o):
        @plsc.parallel_loop(0, _CHUNK, unroll=8)
        def t_body(j):
            w16 = w1_v[pl.ds(k * _CHUNK + (j // _L) * _L, _L)]
            dn = lax.GatherDimensionNumbers(
                offset_dims=(), collapsed_slice_dims=(0,),
                start_index_map=(0,))
            w1 = lax.gather(
                w16, jnp.full((_L, 1), j % _L, jnp.int32), dn, (1,),
                mode=lax.GatherScatterMode.PROMISE_IN_BOUNDS)
            w1b = plsc.pack(w1, w1, format=plsc.PackFormat.INTERLEAVED)
            for u in range(_CHANNELS // 32):
                if u % 4 != 3:
                    a = plsc.bitcast(r0[j, pl.ds(u * _L, _L)], jnp.bfloat16)
                    b = plsc.bitcast(r1[j, pl.ds(u * _L, _L)], jnp.bfloat16)
                    ob16 = a + w1b * (b - a)
                    oe, oo = plsc.unpack(
                        ob16, format=plsc.PackFormat.INTERLEAVED)
                else:
                    a = plsc.bitcast(r0[j, pl.ds(u * _L, _L)], jnp.bfloat16)
                    b = plsc.bitcast(r1[j, pl.ds(u * _L, _L)], jnp.bfloat16)
                    ae, ao = plsc.unpack(a, format=plsc.PackFormat.INTERLEAVED)
                    be, bo = plsc.unpack(b, format=plsc.PackFormat.INTERLEAVED)
                    oe = ae + w1 * (be - ae)
                    oo = ao + w1 * (bo - ao)
                o[j, pl.ds(u * 32, _L)] = oe
                o[j, pl.ds(u * 32 + _L, _L)] = oo

    g_issue(0, r0a, r1a, sg0a, sg1a)

    def pair_body(p, carry):
        k0 = 2 * p
        g_wait(k0, r0a, r1a, sg0a, sg1a)
        g_issue(k0 + 1, r0b, r1b, sg0b, sg1b)

        @pl.when(p > 0)
        def _():
            o_desc(k0 - 2, oa, soa).wait()

        combine(k0, r0a, r1a, oa)
        o_desc(k0, oa, soa).start()

        g_wait(k0 + 1, r0b, r1b, sg0b, sg1b)

        @pl.when(p + 1 < _NPAIR)
        def _():
            g_issue(k0 + 2, r0a, r1a, sg0a, sg1a)

        @pl.when(p > 0)
        def _():
            o_desc(k0 - 1, ob, sob).wait()

        combine(k0 + 1, r0b, r1b, ob)
        o_desc(k0 + 1, ob, sob).start()
        return carry

    lax.fori_loop(0, _NPAIR, pair_body, 0)

    o_desc(_NCHUNK - 2, oa, soa).wait()
    o_desc(_NCHUNK - 1, ob, sob).wait()


def kernel(t, control):
    cperm = (control.astype(jnp.bfloat16)
             .reshape(_STEPS, _CHANNELS // 32, 2, _L)
             .swapaxes(-1, -2))
    cbits = lax.bitcast_convert_type(
        cperm.reshape(_STEPS, _WORDS, 2), jnp.uint32)
    mesh = plsc.VectorSubcoreMesh(core_axis_name="c", subcore_axis_name="s")
    f = pl.kernel(
        _body,
        out_type=jax.ShapeDtypeStruct((_N, _CHANNELS), jnp.float32),
        mesh=mesh,
        compiler_params=pltpu.CompilerParams(needs_layout_passes=False),
        scratch_types=[
            pltpu.VMEM((_PER_W,), jnp.float32),
            pltpu.VMEM((_PER_W,), jnp.int32),
            pltpu.VMEM((_PER_W,), jnp.int32),
            pltpu.VMEM((_PER_W,), jnp.float32),
            pltpu.VMEM((_CHUNK, _WORDS), jnp.uint32),
            pltpu.VMEM((_CHUNK, _WORDS), jnp.uint32),
            pltpu.VMEM((_CHUNK, _WORDS), jnp.uint32),
            pltpu.VMEM((_CHUNK, _WORDS), jnp.uint32),
            pltpu.VMEM((_CHUNK, _CHANNELS), jnp.float32),
            pltpu.VMEM((_CHUNK, _CHANNELS), jnp.float32),
            pltpu.VMEM_SHARED((_STEPS, _WORDS), jnp.uint32),
            pltpu.SemaphoreType.DMA,
            pltpu.SemaphoreType.DMA,
            pltpu.SemaphoreType.DMA,
            pltpu.SemaphoreType.DMA,
            pltpu.SemaphoreType.DMA,
            pltpu.SemaphoreType.DMA,
        ],
    )
    return f(t, cbits)


# speedup vs baseline: 1.0881x; 1.0189x over previous
import jax
import jax.numpy as jnp
from jax import lax
from jax.experimental import pallas as pl
from jax.experimental.pallas import tpu as pltpu
from jax.experimental.pallas import tpu_sc as plsc

_STEPS = 1024
_CHANNELS = 256
_N = 65536
_NC = 2
_NS = 16
_NW = _NC * _NS
_PER_W = _N // _NW
_CHUNK = 64
_NCHUNK = _PER_W // _CHUNK
_NPAIR = _NCHUNK // 2
_L = 16
_WORDS = _CHANNELS // 2


def _body(t_hbm, cbits_hbm, out_hbm,
          t_v, idx0_v, idx1_v, w1_v,
          r0a, r1a, r0b, r1b, r0c, r1c, r0d, r1d, oa, ob, spt,
          sg0a, sg1a, sg0b, sg1b, sg0c, sg1c, sg0d, sg1d, soa, sob):
    sid = lax.axis_index("s")
    wid = sid * _NC + lax.axis_index("c")
    base = wid * _PER_W

    rows_per_tile = _STEPS // _NS
    pltpu.sync_copy(cbits_hbm.at[pl.ds(sid * rows_per_tile, rows_per_tile)],
                    spt.at[pl.ds(sid * rows_per_tile, rows_per_tile)])
    pltpu.sync_copy(t_hbm.at[pl.ds(base, _PER_W)], t_v)

    def idx_body(g, carry):
        t16 = t_v[pl.ds(g * _L, _L)]
        pos = t16 * float(_STEPS - 1)
        i0 = lax.convert_element_type(pos, jnp.int32)
        i0 = jnp.maximum(jnp.minimum(i0, _STEPS - 2), 0)
        frac = pos - lax.convert_element_type(i0, jnp.float32)
        idx0_v[pl.ds(g * _L, _L)] = i0
        idx1_v[pl.ds(g * _L, _L)] = i0 + 1
        w1_v[pl.ds(g * _L, _L)] = frac
        return carry

    lax.fori_loop(0, _PER_W // _L, idx_body, 0)
    plsc.subcore_barrier()

    def g_descs(k, r0, r1, s0, s1):
        d0 = pltpu.make_async_copy(
            spt.at[idx0_v.at[pl.ds(k * _CHUNK, _CHUNK)]], r0, s0)
        d1 = pltpu.make_async_copy(
            spt.at[idx1_v.at[pl.ds(k * _CHUNK, _CHUNK)]], r1, s1)
        return d0, d1

    def g_issue(k, r0, r1, s0, s1):
        d0, d1 = g_descs(k, r0, r1, s0, s1)
        d0.start()
        d1.start()

    def g_wait(k, r0, r1, s0, s1):
        d0, d1 = g_descs(k, r0, r1, s0, s1)
        d0.wait()
        d1.wait()

    def o_desc(k, o, so):
        return pltpu.make_async_copy(
            o, out_hbm.at[pl.ds(base + k * _CHUNK, _CHUNK)], so)

    def combine(k, r0, r1, o):
        @plsc.parallel_loop(0, _CHUNK, unroll=8)
        def t_body(j):
            w16 = w1_v[pl.ds(k * _CHUNK + (j // _L) * _L, _L)]
            dn = lax.GatherDimensionNumbers(
                offset_dims=(), collapsed_slice_dims=(0,),
                start_index_map=(0,))
            w1 = lax.gather(
                w16, jnp.full((_L, 1), j % _L, jnp.int32), dn, (1,),
                mode=lax.GatherScatterMode.PROMISE_IN_BOUNDS)
            w1b = plsc.pack(w1, w1, format=plsc.PackFormat.INTERLEAVED)
            for u in range(_CHANNELS // 32):
                if u % 4 != 3:
                    a = plsc.bitcast(r0[j, pl.ds(u * _L, _L)], jnp.bfloat16)
                    b = plsc.bitcast(r1[j, pl.ds(u * _L, _L)], jnp.bfloat16)
                    ob16 = a + w1b * (b - a)
                    oe, oo = plsc.unpack(
                        ob16, format=plsc.PackFormat.INTERLEAVED)
                else:
                    a = plsc.bitcast(r0[j, pl.ds(u * _L, _L)], jnp.bfloat16)
                    b = plsc.bitcast(r1[j, pl.ds(u * _L, _L)], jnp.bfloat16)
                    ae, ao = plsc.unpack(a, format=plsc.PackFormat.INTERLEAVED)
                    be, bo = plsc.unpack(b, format=plsc.PackFormat.INTERLEAVED)
                    oe = ae + w1 * (be - ae)
                    oo = ao + w1 * (bo - ao)
                o[j, pl.ds(u * 32, _L)] = oe
                o[j, pl.ds(u * 32 + _L, _L)] = oo

    slots = ((r0a, r1a, sg0a, sg1a), (r0b, r1b, sg0b, sg1b),
             (r0c, r1c, sg0c, sg1c), (r0d, r1d, sg0d, sg1d))
    outs = ((oa, soa), (ob, sob))
    for i in range(4):
        g_issue(i, *slots[i])

    def quad_body(p, carry):
        k0 = 4 * p
        for i in range(4):
            k = k0 + i
            r0, r1, s0, s1 = slots[i]
            o, so = outs[i % 2]
            g_wait(k, r0, r1, s0, s1)
            if i >= 2:
                o_desc(k - 2, o, so).wait()
            else:
                @pl.when(p > 0)
                def _():
                    o_desc(k - 2, o, so).wait()
            combine(k, r0, r1, o)
            o_desc(k, o, so).start()

            @pl.when(p + 1 < _NCHUNK // 4)
            def _():
                g_issue(k + 4, r0, r1, s0, s1)
        return carry

    lax.fori_loop(0, _NCHUNK // 4, quad_body, 0)

    o_desc(_NCHUNK - 2, oa, soa).wait()
    o_desc(_NCHUNK - 1, ob, sob).wait()


def kernel(t, control):
    cperm = (control.astype(jnp.bfloat16)
             .reshape(_STEPS, _CHANNELS // 32, 2, _L)
             .swapaxes(-1, -2))
    cbits = lax.bitcast_convert_type(
        cperm.reshape(_STEPS, _WORDS, 2), jnp.uint32)
    mesh = plsc.VectorSubcoreMesh(core_axis_name="c", subcore_axis_name="s")
    f = pl.kernel(
        _body,
        out_type=jax.ShapeDtypeStruct((_N, _CHANNELS), jnp.float32),
        mesh=mesh,
        compiler_params=pltpu.CompilerParams(needs_layout_passes=False),
        scratch_types=[
            pltpu.VMEM((_PER_W,), jnp.float32),
            pltpu.VMEM((_PER_W,), jnp.int32),
            pltpu.VMEM((_PER_W,), jnp.int32),
            pltpu.VMEM((_PER_W,), jnp.float32),
            pltpu.VMEM((_CHUNK, _WORDS), jnp.uint32),
            pltpu.VMEM((_CHUNK, _WORDS), jnp.uint32),
            pltpu.VMEM((_CHUNK, _WORDS), jnp.uint32),
            pltpu.VMEM((_CHUNK, _WORDS), jnp.uint32),
            pltpu.VMEM((_CHUNK, _WORDS), jnp.uint32),
            pltpu.VMEM((_CHUNK, _WORDS), jnp.uint32),
            pltpu.VMEM((_CHUNK, _WORDS), jnp.uint32),
            pltpu.VMEM((_CHUNK, _WORDS), jnp.uint32),
            pltpu.VMEM((_CHUNK, _CHANNELS), jnp.float32),
            pltpu.VMEM((_CHUNK, _CHANNELS), jnp.float32),
            pltpu.VMEM_SHARED((_STEPS, _WORDS), jnp.uint32),
            pltpu.SemaphoreType.DMA,
            pltpu.SemaphoreType.DMA,
            pltpu.SemaphoreType.DMA,
            pltpu.SemaphoreType.DMA,
            pltpu.SemaphoreType.DMA,
            pltpu.SemaphoreType.DMA,
            pltpu.SemaphoreType.DMA,
            pltpu.SemaphoreType.DMA,
            pltpu.SemaphoreType.DMA,
            pltpu.SemaphoreType.DMA,
        ],
    )
    return f(t, cbits)
